# Initial kernel scaffold; baseline (speedup 1.0000x reference)
import jax
import jax.numpy as jnp
from jax.experimental import pallas as pl


def kernel(c_feat, g_feat, edge_index):
    raise NotImplementedError("write your pallas kernel here")


# speedup vs baseline: 2.5679x; 2.5679x over previous
import dataclasses
import functools

import jax
import jax.numpy as jnp
from jax import lax
from jax.experimental import pallas as pl
from jax.experimental.pallas import tpu as pltpu
from jax.experimental.pallas import tpu_sc as plsc

L = 16
N_CORES = 2
N_SUBCORES = 16
N_WORKERS = N_CORES * N_SUBCORES
CHUNK = 80


def _sc_edge_dot(c_feat, g_feat, src, dst, n_edges, d_feat):
    per_w = n_edges // N_WORKERS
    n_steps = per_w // CHUNK
    n_groups = CHUNK // L
    n_dch = d_feat // L

    mesh = plsc.VectorSubcoreMesh(core_axis_name="c", subcore_axis_name="s")

    cp = pltpu.CompilerParams()
    if "needs_layout_passes" in pltpu.CompilerParams.__dataclass_fields__:
        cp = dataclasses.replace(cp, needs_layout_passes=False)

    @functools.partial(
        pl.kernel,
        compiler_params=cp,
        out_type=jax.ShapeDtypeStruct((n_edges,), jnp.float32),
        mesh=mesh,
        scratch_types=[
            pltpu.VMEM((CHUNK,), jnp.int32),
            pltpu.VMEM((CHUNK,), jnp.int32),
            pltpu.VMEM((CHUNK, d_feat), jnp.float32),
            pltpu.VMEM((CHUNK, d_feat), jnp.float32),
            pltpu.VMEM((CHUNK,), jnp.float32),
            pltpu.SemaphoreType.DMA,
            pltpu.SemaphoreType.DMA,
        ],
    )
    def sc_kernel(c_hbm, g_hbm, src_hbm, dst_hbm, out_hbm,
                  si_v, di_v, u_v, v_v, o_v, sem_u, sem_v):
        wid = lax.axis_index("s") * N_CORES + lax.axis_index("c")
        base_w = wid * per_w

        @pl.loop(0, n_steps)
        def _(step):
            base = base_w + step * CHUNK
            pltpu.sync_copy(src_hbm.at[pl.ds(base, CHUNK)], si_v)
            pltpu.sync_copy(dst_hbm.at[pl.ds(base, CHUNK)], di_v)
            cp_u = pltpu.async_copy(c_hbm.at[si_v], u_v, sem_u)
            cp_v = pltpu.async_copy(g_hbm.at[di_v], v_v, sem_v)
            cp_u.wait()
            cp_v.wait()

            @pl.loop(0, n_groups)
            def _(g):
                dots = jnp.zeros((L,), jnp.float32)
                for e in range(L):
                    row = g * L + e
                    acc = u_v[row, pl.ds(0, L)] * v_v[row, pl.ds(0, L)]
                    for c in range(1, n_dch):
                        acc = acc + (u_v[row, pl.ds(c * L, L)]
                                     * v_v[row, pl.ds(c * L, L)])
                    s = jnp.sum(acc)
                    sel = lax.iota(jnp.int32, L) == e
                    dots = jnp.where(sel, s, dots)
                o_v[pl.ds(g * L, L)] = 1.0 / (1.0 + jnp.exp(-dots))

            pltpu.sync_copy(o_v, out_hbm.at[pl.ds(base, CHUNK)])

    return sc_kernel(c_feat, g_feat, src, dst)


def kernel(c_feat, g_feat, edge_index):
    n_nodes, d_feat = c_feat.shape
    n_edges = edge_index.shape[1]
    src = edge_index[0].astype(jnp.int32)
    dst = edge_index[1].astype(jnp.int32)
    scores = _sc_edge_dot(c_feat, g_feat, src, dst, n_edges, d_feat)
    return scores.reshape(n_edges, 1)


# speedup vs baseline: 4.0536x; 1.5786x over previous
import dataclasses
import functools

import jax
import jax.numpy as jnp
from jax import lax
from jax.experimental import pallas as pl
from jax.experimental.pallas import tpu as pltpu
from jax.experimental.pallas import tpu_sc as plsc

L = 16
N_CORES = 2
N_SUBCORES = 16
N_WORKERS = N_CORES * N_SUBCORES
CHUNK = 80


def _sc_edge_dot(c_feat, g_feat, src, dst, n_edges, d_feat):
    per_w = n_edges // N_WORKERS
    n_steps = per_w // CHUNK
    n_pairs = (n_steps - 1) // 2
    n_groups = CHUNK // L
    n_dch = d_feat // L

    mesh = plsc.VectorSubcoreMesh(core_axis_name="c", subcore_axis_name="s")

    cp = pltpu.CompilerParams()
    if "needs_layout_passes" in pltpu.CompilerParams.__dataclass_fields__:
        cp = dataclasses.replace(cp, needs_layout_passes=False)

    @functools.partial(
        pl.kernel,
        compiler_params=cp,
        out_type=jax.ShapeDtypeStruct((n_edges,), jnp.float32),
        mesh=mesh,
        scratch_types=[
            pltpu.VMEM((per_w,), jnp.int32),
            pltpu.VMEM((per_w,), jnp.int32),
            pltpu.VMEM((CHUNK, d_feat), jnp.float32),
            pltpu.VMEM((CHUNK, d_feat), jnp.float32),
            pltpu.VMEM((CHUNK, d_feat), jnp.float32),
            pltpu.VMEM((CHUNK, d_feat), jnp.float32),
            pltpu.VMEM((per_w,), jnp.float32),
            pltpu.SemaphoreType.DMA,
            pltpu.SemaphoreType.DMA,
        ],
    )
    def sc_kernel(c_hbm, g_hbm, src_hbm, dst_hbm, out_hbm,
                  si_v, di_v, u_a, v_a, u_b, v_b, o_v, sem_a, sem_b):
        wid = lax.axis_index("s") * N_CORES + lax.axis_index("c")
        base_w = wid * per_w
        pltpu.sync_copy(src_hbm.at[pl.ds(base_w, per_w)], si_v)
        pltpu.sync_copy(dst_hbm.at[pl.ds(base_w, per_w)], di_v)

        def start(step, u_buf, v_buf, sem):
            off = step * CHUNK
            pltpu.async_copy(c_hbm.at[si_v.at[pl.ds(off, CHUNK)]], u_buf, sem)
            pltpu.async_copy(g_hbm.at[di_v.at[pl.ds(off, CHUNK)]], v_buf, sem)

        def wait(u_buf, v_buf, sem):
            pltpu.make_async_copy(c_hbm.at[pl.ds(0, CHUNK)], u_buf, sem).wait()
            pltpu.make_async_copy(c_hbm.at[pl.ds(0, CHUNK)], v_buf, sem).wait()

        def compute(step, u_buf, v_buf):
            obase = step * CHUNK

            @pl.loop(0, n_groups)
            def _(g):
                dots = jnp.zeros((L,), jnp.float32)
                for e in range(L):
                    row = g * L + e
                    acc = u_buf[row, pl.ds(0, L)] * v_buf[row, pl.ds(0, L)]
                    for c in range(1, n_dch):
                        acc = acc + (u_buf[row, pl.ds(c * L, L)]
                                     * v_buf[row, pl.ds(c * L, L)])
                    s = jnp.sum(acc)
                    sel = lax.iota(jnp.int32, L) == e
                    dots = jnp.where(sel, s, dots)
                o_v[pl.ds(obase + g * L, L)] = 1.0 / (1.0 + jnp.exp(-dots))

        start(0, u_a, v_a, sem_a)

        @pl.loop(0, n_pairs)
        def _(p):
            s = 1 + 2 * p
            start(s, u_b, v_b, sem_b)
            wait(u_a, v_a, sem_a)
            compute(s - 1, u_a, v_a)
            start(s + 1, u_a, v_a, sem_a)
            wait(u_b, v_b, sem_b)
            compute(s, u_b, v_b)

        wait(u_a, v_a, sem_a)
        compute(n_steps - 1, u_a, v_a)
        pltpu.sync_copy(o_v, out_hbm.at[pl.ds(base_w, per_w)])

    return sc_kernel(c_feat, g_feat, src, dst)


def kernel(c_feat, g_feat, edge_index):
    n_nodes, d_feat = c_feat.shape
    n_edges = edge_index.shape[1]
    src = edge_index[0].astype(jnp.int32)
    dst = edge_index[1].astype(jnp.int32)
    scores = _sc_edge_dot(c_feat, g_feat, src, dst, n_edges, d_feat)
    return scores.reshape(n_edges, 1)
